# baseline (speedup 1.0000x reference)
import jax
import jax.numpy as jnp
from jax import lax
from jax.experimental import pallas as pl
from jax.experimental.pallas import tpu as pltpu
from jax.experimental.pallas import tpu_sc as plsc

VOCAB = 1000000
EMBED_DIM = 64
B = 16384
L = 50
N = B * L


TC_BS = 32768
TC_GRID = pl.cdiv(VOCAB, TC_BS)
VPAD = TC_GRID * TC_BS


def _precompute_body(tab_ref, w_ref, b_ref, out_ref):
    t = tab_ref[...]
    w = w_ref[...]
    zt = jnp.dot(w, t, preferred_element_type=jnp.float32)
    out_ref[...] = jax.nn.sigmoid(zt + b_ref[0, 0]).reshape(TC_BS)


def _precompute(table, W, b):
    t_t = table.T
    return pl.pallas_call(
        _precompute_body,
        grid=(TC_GRID,),
        in_specs=[
            pl.BlockSpec((EMBED_DIM, TC_BS), lambda i: (0, i)),
            pl.BlockSpec((1, EMBED_DIM), lambda i: (0, 0)),
            pl.BlockSpec(memory_space=pltpu.SMEM),
        ],
        out_specs=pl.BlockSpec((TC_BS,), lambda i: (i,)),
        out_shape=jax.ShapeDtypeStruct((VPAD,), jnp.float32),
    )(t_t, W.reshape(1, EMBED_DIM), b.reshape(1, 1))


_NC, _NS = 2, 16
_NW = _NC * _NS
_PER_W = N // _NW


def _gather_body(v_hbm, idx_hbm, out_hbm, idx_v, val_v, sem):
    wid = lax.axis_index("s") * _NC + lax.axis_index("c")
    base = wid * _PER_W
    pltpu.sync_copy(idx_hbm.at[pl.ds(base, _PER_W)], idx_v)
    pltpu.async_copy(v_hbm.at[idx_v], val_v, sem).wait()
    pltpu.sync_copy(val_v, out_hbm.at[pl.ds(base, _PER_W)])


def _sc_gather(v_flat, idx_flat):
    mesh = plsc.VectorSubcoreMesh(core_axis_name="c", subcore_axis_name="s")
    return pl.kernel(
        _gather_body,
        mesh=mesh,
        out_type=jax.ShapeDtypeStruct((N,), jnp.float32),
        scratch_types=[
            pltpu.VMEM((_PER_W,), jnp.int32),
            pltpu.VMEM((_PER_W,), jnp.float32),
            pltpu.SemaphoreType.DMA,
        ],
    )(v_flat, idx_flat)


def kernel(x, table, W, b):
    v = _precompute(table, W, b)
    idx = x.T.reshape(N).astype(jnp.int32)
    out = _sc_gather(v, idx)
    return out.reshape(L, B, 1).transpose((1, 0, 2))
